# baseline (speedup 1.0000x reference)
import functools

import jax
import jax.numpy as jnp
from jax import lax
from jax.experimental import pallas as pl
from jax.experimental.pallas import tpu as pltpu
from jax.experimental.pallas import tpu_sc as plsc

_N_WORKERS = 32
_LANE = 128
_SUB = 8


@functools.lru_cache(maxsize=None)
def _make_relayout(vocab: int, dim: int):
    qrows = vocab * dim // _LANE
    per_q = _LANE // dim
    n_full = vocab // _LANE
    tail = vocab - n_full * _LANE
    n_iter = (n_full + _N_WORKERS - 1) // _N_WORKERS
    pitch = per_q * dim + 4
    mesh = plsc.VectorSubcoreMesh(core_axis_name="c", subcore_axis_name="s")

    @functools.partial(
        pl.kernel,
        mesh=mesh,
        out_type=jax.ShapeDtypeStruct((qrows, _LANE), jnp.float32),
        scratch_types=[
            pltpu.VMEM((dim, _LANE), jnp.float32),
            pltpu.VMEM((_LANE // per_q, pitch), jnp.float32),
            pltpu.VMEM((tail * dim // _LANE, _LANE), jnp.float32),
        ]
        + [pltpu.SemaphoreType.DMA] * 2,
        compiler_params=pltpu.CompilerParams(
            use_tc_tiling_on_sc=True, needs_layout_passes=False
        ),
    )
    def relayout_kernel(tbl_t, tail_q, out_hbm, tb, rb, tq, si, so):
        wid = lax.axis_index("s") * 2 + lax.axis_index("c")
        lane = lax.iota(jnp.int32, 16)

        def block(vt):
            pltpu.make_async_copy(
                tbl_t.at[:, pl.ds(vt * _LANE, _LANE)], tb, si
            ).start()
            pltpu.make_async_copy(
                tbl_t.at[:, pl.ds(vt * _LANE, _LANE)], tb, si
            ).wait()
            for j in range(_LANE // 16):
                vv = lane + j * 16
                qv = vv // per_q
                wb = (vv % per_q) * dim
                for d in range(dim):
                    x = tb[d, pl.ds(j * 16, 16)]
                    plsc.store_scatter(rb, [qv, wb + d], x)
            nq = _LANE // per_q
            pltpu.make_async_copy(
                rb.at[:, 0:_LANE],
                out_hbm.at[pl.ds(vt * nq, nq), :],
                so,
            ).start()
            pltpu.make_async_copy(
                rb.at[:, 0:_LANE],
                out_hbm.at[pl.ds(vt * nq, nq), :],
                so,
            ).wait()

        def step(i, carry):
            vt = wid + i * _N_WORKERS

            @pl.when(vt < n_full)
            def _():
                block(vt)

            return carry

        lax.fori_loop(0, n_iter, step, 0)

        if tail:
            @pl.when(wid == 0)
            def _():
                nq = tail * dim // _LANE
                pltpu.make_async_copy(tail_q, tq, si).start()
                pltpu.make_async_copy(tail_q, tq, si).wait()
                pltpu.make_async_copy(
                    tq, out_hbm.at[pl.ds(n_full * (_LANE // per_q), nq), :], so
                ).start()
                pltpu.make_async_copy(
                    tq, out_hbm.at[pl.ds(n_full * (_LANE // per_q), nq), :], so
                ).wait()

    return relayout_kernel


@functools.lru_cache(maxsize=None)
def _make_gather(hist: int, batch: int, dim: int):
    bw = batch // _N_WORKERS
    assert bw == _LANE
    td = dim // _SUB
    n_groups = hist // 2
    pitch = _LANE + 1
    mesh = plsc.VectorSubcoreMesh(core_axis_name="c", subcore_axis_name="s")

    @functools.partial(
        pl.kernel,
        mesh=mesh,
        out_type=jax.ShapeDtypeStruct(
            (hist, td, _N_WORKERS, _SUB, _LANE), jnp.float32
        ),
        scratch_types=[
            pltpu.VMEM((hist, bw), jnp.int32),
            pltpu.VMEM((bw, dim), jnp.float32),
            pltpu.VMEM((bw, dim), jnp.float32),
            pltpu.VMEM((td, _SUB, pitch), jnp.float32),
            pltpu.VMEM((td, _SUB, pitch), jnp.float32),
        ]
        + [pltpu.SemaphoreType.DMA] * 5,
        compiler_params=pltpu.CompilerParams(
            use_tc_tiling_on_sc=False, needs_layout_passes=False
        ),
    )
    def gather_kernel(ids_hbm, table_hbm, out_hbm, idx_v, rows0, rows1,
                      tr0, tr1, sem_i, g0, g1, o0, o1):
        wid = lax.axis_index("s") * 2 + lax.axis_index("c")
        b0 = wid * bw

        lane = lax.iota(jnp.int32, 16)
        td_lo, di_lo = lane // _SUB, lane % _SUB
        td_hi = td_lo + 16 // _SUB

        def transpose(rows, tr):
            for r in range(bw):
                rv = jnp.full((16,), r, jnp.int32)
                plsc.store_scatter(tr, [td_lo, di_lo, rv], rows[r, 0:16])
                plsc.store_scatter(tr, [td_hi, di_lo, rv], rows[r, 16:32])

        def gather_copy(h, rows, sem):
            return pltpu.make_async_copy(
                table_hbm.at[idx_v.at[h]], rows, sem
            )

        def out_copy(h, tr, sem):
            return pltpu.make_async_copy(
                tr.at[:, :, 0:bw], out_hbm.at[h, :, wid, :, :], sem
            )

        pltpu.make_async_copy(
            ids_hbm.at[:, pl.ds(b0, bw)], idx_v, sem_i
        ).start()
        pltpu.make_async_copy(
            ids_hbm.at[:, pl.ds(b0, bw)], idx_v, sem_i
        ).wait()
        gather_copy(0, rows0, g0).start()

        def group(g, carry):
            h0 = 2 * g
            h1 = h0 + 1
            gather_copy(h0, rows0, g0).wait()
            gather_copy(h1, rows1, g1).start()

            @pl.when(g > 0)
            def _():
                out_copy(h0, tr0, o0).wait()

            transpose(rows0, tr0)
            out_copy(h0, tr0, o0).start()

            @pl.when(g < n_groups - 1)
            def _():
                gather_copy(h0 + 2, rows0, g0).start()

            gather_copy(h1, rows1, g1).wait()

            @pl.when(g > 0)
            def _():
                out_copy(h1, tr1, o1).wait()

            transpose(rows1, tr1)
            out_copy(h1, tr1, o1).start()
            return carry

        lax.fori_loop(0, n_groups, group, 0)
        out_copy(hist - 2, tr0, o0).wait()
        out_copy(hist - 1, tr1, o1).wait()

    return gather_kernel


def kernel(input_ids, table):
    batch, hist = input_ids.shape
    vocab, dim = table.shape
    ids_t = jnp.transpose(input_ids).astype(jnp.int32)
    tbl_t = jnp.transpose(table)
    n_full = vocab // _LANE
    tail_q = jnp.reshape(
        table[n_full * _LANE:, :], ((vocab - n_full * _LANE) * dim // _LANE, _LANE)
    )
    compact = _make_relayout(vocab, dim)(tbl_t, tail_q)
    tbl_rm = jnp.reshape(compact, (vocab, dim))
    out_tiles = _make_gather(hist, batch, dim)(ids_t, tbl_rm)
    return jnp.transpose(out_tiles, (2, 4, 0, 1, 3)).reshape(batch, hist, dim)


# speedup vs baseline: 1.3027x; 1.3027x over previous
import functools

import jax
import jax.numpy as jnp
from jax import lax
from jax.experimental import pallas as pl
from jax.experimental.pallas import tpu as pltpu
from jax.experimental.pallas import tpu_sc as plsc

_N_WORKERS = 32
_LANE = 128
_SUB = 8


@functools.lru_cache(maxsize=None)
def _make_relayout(vocab: int, dim: int):
    qrows = vocab * dim // _LANE
    per_q = _LANE // dim
    n_full = vocab // _LANE
    tail = vocab - n_full * _LANE
    n_iter = (n_full + _N_WORKERS - 1) // _N_WORKERS
    pitch = per_q * dim + 4
    mesh = plsc.VectorSubcoreMesh(core_axis_name="c", subcore_axis_name="s")

    @functools.partial(
        pl.kernel,
        mesh=mesh,
        out_type=jax.ShapeDtypeStruct((qrows, _LANE), jnp.float32),
        scratch_types=[
            pltpu.VMEM((dim, _LANE), jnp.float32),
            pltpu.VMEM((dim, _LANE), jnp.float32),
            pltpu.VMEM((_LANE // per_q, pitch), jnp.float32),
            pltpu.VMEM((_LANE // per_q, pitch), jnp.float32),
            pltpu.VMEM((tail * dim // _LANE, _LANE), jnp.float32),
        ]
        + [pltpu.SemaphoreType.DMA] * 4,
        compiler_params=pltpu.CompilerParams(
            use_tc_tiling_on_sc=True, needs_layout_passes=False
        ),
    )
    def relayout_kernel(tbl_t, tail_q, out_hbm, tb0, tb1, rb0, rb1, tq,
                        i0, i1, o0, o1):
        wid = lax.axis_index("s") * 2 + lax.axis_index("c")
        lane = lax.iota(jnp.int32, 16)
        nq = _LANE // per_q
        tbb, rbb = (tb0, tb1), (rb0, rb1)
        isem, osem = (i0, i1), (o0, o1)

        def in_copy(vt, b):
            return pltpu.make_async_copy(
                tbl_t.at[:, pl.ds(vt * _LANE, _LANE)], tbb[b], isem[b]
            )

        def out_copy(vt, b):
            return pltpu.make_async_copy(
                rbb[b].at[:, 0:_LANE], out_hbm.at[pl.ds(vt * nq, nq), :],
                osem[b],
            )

        def transpose(b):
            tb, rb = tbb[b], rbb[b]
            for j in range(_LANE // 16):
                vv = lane + j * 16
                qv = vv // per_q
                wb = (vv % per_q) * dim
                for d in range(dim):
                    plsc.store_scatter(rb, [qv, wb + d], tb[d, pl.ds(j * 16, 16)])

        in_copy(wid, 0).start()
        in_copy(wid + _N_WORKERS, 1).start()

        def half(k, ih, b):
            vt = wid + ih * _N_WORKERS

            @pl.when((k > 0) & (vt - 2 * _N_WORKERS < n_full))
            def _():
                out_copy(vt - 2 * _N_WORKERS, b).wait()

            @pl.when(vt < n_full)
            def _():
                in_copy(vt, b).wait()
                transpose(b)
                out_copy(vt, b).start()

                @pl.when(vt + 2 * _N_WORKERS < n_full)
                def _():
                    in_copy(vt + 2 * _N_WORKERS, b).start()

        def step(k, carry):
            half(k, 2 * k, 0)
            half(k, 2 * k + 1, 1)
            return carry

        n_pairs = (n_iter + 1) // 2
        lax.fori_loop(0, n_pairs, step, 0)

        last0 = 2 * (n_pairs - 1)
        last1 = last0 + 1
        for ih, b in ((last0, 0), (last1, 1)):
            vt = wid + ih * _N_WORKERS

            @pl.when(vt < n_full)
            def _():
                out_copy(vt, b).wait()

        if tail:
            @pl.when(wid == 0)
            def _():
                pltpu.make_async_copy(tail_q, tq, i0).start()
                pltpu.make_async_copy(tail_q, tq, i0).wait()
                pltpu.make_async_copy(
                    tq, out_hbm.at[pl.ds(n_full * nq, tail * dim // _LANE), :],
                    o0,
                ).start()
                pltpu.make_async_copy(
                    tq, out_hbm.at[pl.ds(n_full * nq, tail * dim // _LANE), :],
                    o0,
                ).wait()

    return relayout_kernel


@functools.lru_cache(maxsize=None)
def _make_gather(hist: int, batch: int, dim: int):
    bw = batch // _N_WORKERS
    assert bw == _LANE
    td = dim // _SUB
    n_groups = hist // 2
    pitch = _LANE + 1
    mesh = plsc.VectorSubcoreMesh(core_axis_name="c", subcore_axis_name="s")

    @functools.partial(
        pl.kernel,
        mesh=mesh,
        out_type=jax.ShapeDtypeStruct(
            (hist, td, _N_WORKERS, _SUB, _LANE), jnp.float32
        ),
        scratch_types=[
            pltpu.VMEM((hist, bw), jnp.int32),
            pltpu.VMEM((bw, dim), jnp.float32),
            pltpu.VMEM((bw, dim), jnp.float32),
            pltpu.VMEM((td, _SUB, pitch), jnp.float32),
            pltpu.VMEM((td, _SUB, pitch), jnp.float32),
        ]
        + [pltpu.SemaphoreType.DMA] * 5,
        compiler_params=pltpu.CompilerParams(
            use_tc_tiling_on_sc=False, needs_layout_passes=False
        ),
    )
    def gather_kernel(ids_hbm, table_hbm, out_hbm, idx_v, rows0, rows1,
                      tr0, tr1, sem_i, g0, g1, o0, o1):
        wid = lax.axis_index("s") * 2 + lax.axis_index("c")
        b0 = wid * bw

        lane = lax.iota(jnp.int32, 16)
        td_lo, di_lo = lane // _SUB, lane % _SUB
        td_hi = td_lo + 16 // _SUB

        def transpose(rows, tr):
            for r in range(bw):
                rv = jnp.full((16,), r, jnp.int32)
                plsc.store_scatter(tr, [td_lo, di_lo, rv], rows[r, 0:16])
                plsc.store_scatter(tr, [td_hi, di_lo, rv], rows[r, 16:32])

        def gather_copy(h, rows, sem):
            return pltpu.make_async_copy(
                table_hbm.at[idx_v.at[h]], rows, sem
            )

        def out_copy(h, tr, sem):
            return pltpu.make_async_copy(
                tr.at[:, :, 0:bw], out_hbm.at[h, :, wid, :, :], sem
            )

        pltpu.make_async_copy(
            ids_hbm.at[:, pl.ds(b0, bw)], idx_v, sem_i
        ).start()
        pltpu.make_async_copy(
            ids_hbm.at[:, pl.ds(b0, bw)], idx_v, sem_i
        ).wait()
        gather_copy(0, rows0, g0).start()

        def group(g, carry):
            h0 = 2 * g
            h1 = h0 + 1
            gather_copy(h0, rows0, g0).wait()
            gather_copy(h1, rows1, g1).start()

            @pl.when(g > 0)
            def _():
                out_copy(h0, tr0, o0).wait()

            transpose(rows0, tr0)
            out_copy(h0, tr0, o0).start()

            @pl.when(g < n_groups - 1)
            def _():
                gather_copy(h0 + 2, rows0, g0).start()

            gather_copy(h1, rows1, g1).wait()

            @pl.when(g > 0)
            def _():
                out_copy(h1, tr1, o1).wait()

            transpose(rows1, tr1)
            out_copy(h1, tr1, o1).start()
            return carry

        lax.fori_loop(0, n_groups, group, 0)
        out_copy(hist - 2, tr0, o0).wait()
        out_copy(hist - 1, tr1, o1).wait()

    return gather_kernel


def kernel(input_ids, table):
    batch, hist = input_ids.shape
    vocab, dim = table.shape
    ids_t = jnp.transpose(input_ids).astype(jnp.int32)
    tbl_t = jnp.transpose(table)
    n_full = vocab // _LANE
    tail_q = jnp.reshape(
        table[n_full * _LANE:, :], ((vocab - n_full * _LANE) * dim // _LANE, _LANE)
    )
    compact = _make_relayout(vocab, dim)(tbl_t, tail_q)
    tbl_rm = jnp.reshape(compact, (vocab, dim))
    out_tiles = _make_gather(hist, batch, dim)(ids_t, tbl_rm)
    return jnp.transpose(out_tiles, (2, 4, 0, 1, 3)).reshape(batch, hist, dim)


# speedup vs baseline: 1.5423x; 1.1839x over previous
import functools

import jax
import jax.numpy as jnp
from jax import lax
from jax.experimental import pallas as pl
from jax.experimental.pallas import tpu as pltpu
from jax.experimental.pallas import tpu_sc as plsc

_N_WORKERS = 32
_LANE = 128
_SUB = 8


@functools.lru_cache(maxsize=None)
def _make_gather(hist: int, batch: int, dim: int):
    bw = batch // _N_WORKERS
    assert bw == _LANE
    td = dim // _SUB
    n_groups = hist // 2
    pitch = _LANE + 1
    mesh = plsc.VectorSubcoreMesh(core_axis_name="c", subcore_axis_name="s")

    @functools.partial(
        pl.kernel,
        mesh=mesh,
        out_type=jax.ShapeDtypeStruct(
            (hist, td, _N_WORKERS, _SUB, _LANE), jnp.float32
        ),
        scratch_types=[
            pltpu.VMEM((hist, bw), jnp.int32),
            pltpu.VMEM((bw, dim), jnp.float32),
            pltpu.VMEM((bw, dim), jnp.float32),
            pltpu.VMEM((td, _SUB, pitch), jnp.float32),
            pltpu.VMEM((td, _SUB, pitch), jnp.float32),
        ]
        + [pltpu.SemaphoreType.DMA] * 5,
        compiler_params=pltpu.CompilerParams(
            use_tc_tiling_on_sc=False, needs_layout_passes=False
        ),
    )
    def gather_kernel(ids_hbm, table_hbm, out_hbm, idx_v, rows0, rows1,
                      tr0, tr1, sem_i, g0, g1, o0, o1):
        wid = lax.axis_index("s") * 2 + lax.axis_index("c")
        b0 = wid * bw

        lane = lax.iota(jnp.int32, 16)
        td_lo, di_lo = lane // _SUB, lane % _SUB
        td_hi = td_lo + 16 // _SUB

        def transpose(rows, tr):
            for r in range(bw):
                rv = jnp.full((16,), r, jnp.int32)
                plsc.store_scatter(tr, [td_lo, di_lo, rv], rows[r, 0:16])
                plsc.store_scatter(tr, [td_hi, di_lo, rv], rows[r, 16:32])

        def gather_copy(h, rows, sem):
            return pltpu.make_async_copy(
                table_hbm.at[idx_v.at[h]], rows, sem
            )

        def out_copy(h, tr, sem):
            return pltpu.make_async_copy(
                tr.at[:, :, 0:bw], out_hbm.at[h, :, wid, :, :], sem
            )

        pltpu.make_async_copy(
            ids_hbm.at[:, pl.ds(b0, bw)], idx_v, sem_i
        ).start()
        pltpu.make_async_copy(
            ids_hbm.at[:, pl.ds(b0, bw)], idx_v, sem_i
        ).wait()
        gather_copy(0, rows0, g0).start()

        def group(g, carry):
            h0 = 2 * g
            h1 = h0 + 1
            gather_copy(h0, rows0, g0).wait()
            gather_copy(h1, rows1, g1).start()

            @pl.when(g > 0)
            def _():
                out_copy(h0, tr0, o0).wait()

            transpose(rows0, tr0)
            out_copy(h0, tr0, o0).start()

            @pl.when(g < n_groups - 1)
            def _():
                gather_copy(h0 + 2, rows0, g0).start()

            gather_copy(h1, rows1, g1).wait()

            @pl.when(g > 0)
            def _():
                out_copy(h1, tr1, o1).wait()

            transpose(rows1, tr1)
            out_copy(h1, tr1, o1).start()
            return carry

        lax.fori_loop(0, n_groups, group, 0)
        out_copy(hist - 2, tr0, o0).wait()
        out_copy(hist - 1, tr1, o1).wait()

    return gather_kernel


def kernel(input_ids, table):
    batch, hist = input_ids.shape
    dim = table.shape[1]
    ids_t = jnp.transpose(input_ids).astype(jnp.int32)
    out_tiles = _make_gather(hist, batch, dim)(ids_t, table)
    return jnp.transpose(out_tiles, (2, 4, 0, 1, 3)).reshape(batch, hist, dim)
